# Initial kernel scaffold; baseline (speedup 1.0000x reference)
import jax
import jax.numpy as jnp
from jax.experimental import pallas as pl


def kernel(codec, codec_lengths, embed):
    raise NotImplementedError("write your pallas kernel here")


# speedup vs baseline: 2.8611x; 2.8611x over previous
import functools

import jax
import jax.numpy as jnp
from jax import lax
from jax.experimental import pallas as pl
from jax.experimental.pallas import tpu as pltpu
from jax.experimental.pallas import tpu_sc as plsc

NQ = 8
DIM = 256
CBS = 1024
BZ = 16
TT = 2048
NW = 32
TOK_PER_W = BZ * TT // NW
C = 16
NCHUNK = TOK_PER_W // C
L = 16


def _sc_body(codec_hbm, nv_hbm, emb_hbm, out_hbm, idx_v, rows_v, out_v,
             len_v, sem):
    wid = lax.axis_index("s") * 2 + lax.axis_index("c")
    tok0 = wid * TOK_PER_W

    pltpu.sync_copy(nv_hbm.at[wid], len_v)
    nvalid = len_v[...]

    shift = (lax.iota(jnp.int32, L) & (NQ - 1)) * CBS

    def chunk(g, _):
        tok_base = tok0 + g * C
        pltpu.sync_copy(codec_hbm.at[pl.ds(tok_base * NQ, C * NQ)], idx_v)
        for j in range(C * NQ // L):
            idx_v[pl.ds(j * L, L)] = idx_v[pl.ds(j * L, L)] + shift
        pltpu.async_copy(emb_hbm.at[idx_v], rows_v, sem).wait()

        def tok(i, _):
            tpos = g * C + i
            m = jnp.where(tpos < nvalid, 1.0, 0.0).astype(jnp.float32)
            for d in range(DIM // L):
                acc = rows_v[i * NQ, pl.ds(d * L, L)]
                for q in range(1, NQ):
                    acc = acc + rows_v[i * NQ + q, pl.ds(d * L, L)]
                out_v[i, pl.ds(d * L, L)] = acc * m
            return 0

        lax.fori_loop(0, C, tok, 0)
        pltpu.sync_copy(out_v, out_hbm.at[pl.ds(tok_base, C)])
        return 0

    lax.fori_loop(0, NCHUNK, chunk, 0)


@jax.jit
def _decode(codec_flat, nv_bcast, emb):
    mesh = plsc.VectorSubcoreMesh(core_axis_name="c", subcore_axis_name="s")
    f = functools.partial(
        pl.kernel,
        mesh=mesh,
        out_type=jax.ShapeDtypeStruct((BZ * TT, DIM), jnp.float32),
        scratch_types=[
            pltpu.VMEM((C * NQ,), jnp.int32),
            pltpu.VMEM((C * NQ, DIM), jnp.float32),
            pltpu.VMEM((C, DIM), jnp.float32),
            pltpu.VMEM((L,), jnp.int32),
            pltpu.SemaphoreType.DMA,
        ],
    )(_sc_body)
    return f(codec_flat, nv_bcast, emb)


def kernel(codec, codec_lengths, embed):
    if codec.ndim == 2:
        codec = codec[:, :, None]
    codec_flat = codec.reshape(-1).astype(jnp.int32)
    emb = embed.reshape(NQ * CBS, DIM)
    w = jnp.arange(NW, dtype=jnp.int32)
    nvalid = jnp.clip(codec_lengths[w // 2] - (w % 2) * TOK_PER_W,
                      0, TOK_PER_W).astype(jnp.int32)
    nv_bcast = jnp.broadcast_to(nvalid[:, None], (NW, L))
    out = _decode(codec_flat, nv_bcast, emb)
    return (out.reshape(BZ, TT, DIM), codec_lengths)
